# Initial kernel scaffold; baseline (speedup 1.0000x reference)
import jax
import jax.numpy as jnp
from jax.experimental import pallas as pl


def kernel(x, Wg, bg, W1, W3, W2):
    raise NotImplementedError("write your pallas kernel here")


# speedup vs baseline: 1.0673x; 1.0673x over previous
import functools

import jax
import jax.numpy as jnp
from jax.experimental import pallas as pl
from jax.experimental.pallas import tpu as pltpu

E = 8
TOPK = 2
LANES = 128


def _moe_dense_kernel(x_ref, wg_ref, bg_ref, w1_ref, w3_ref, w2_ref,
                      out_ref, g_scr, *, n_e, n_f):
    e = pl.program_id(1)
    f = pl.program_id(2)
    x = x_ref[...]
    T = x.shape[0]

    @pl.when(jnp.logical_and(e == 0, f == 0))
    def _gate():
        logits = jnp.dot(x, wg_ref[...], preferred_element_type=jnp.float32)
        logits = logits + bg_ref[...]
        lane = jax.lax.broadcasted_iota(jnp.int32, (T, LANES), 1)
        neg = jnp.float32(-1e30)
        logits = jnp.where(lane < n_e, logits, neg)
        m = jnp.max(logits, axis=1, keepdims=True)
        ex = jnp.exp(logits - m)
        probs = ex / jnp.sum(ex, axis=1, keepdims=True)
        i1 = jnp.argmax(probs, axis=1)[:, None]
        p1 = jnp.max(probs, axis=1, keepdims=True)
        probs2 = jnp.where(lane == i1, jnp.float32(-1.0), probs)
        i2 = jnp.argmax(probs2, axis=1)[:, None]
        p2 = jnp.max(probs2, axis=1, keepdims=True)
        wsum = p1 + p2
        g = (jnp.where(lane == i1, p1, 0.0) + jnp.where(lane == i2, p2, 0.0)) / wsum
        g_scr[...] = g

    lane = jax.lax.broadcasted_iota(jnp.int32, (T, LANES), 1)
    g_e = jnp.sum(jnp.where(lane == e, g_scr[...], 0.0), axis=1, keepdims=True)

    a = jnp.dot(x, w1_ref[0], preferred_element_type=jnp.float32)
    b = jnp.dot(x, w3_ref[0], preferred_element_type=jnp.float32)
    h = (a * jax.lax.logistic(a)) * b
    y = jnp.dot(h, w2_ref[0], preferred_element_type=jnp.float32)

    @pl.when(jnp.logical_and(e == 0, f == 0))
    def _init():
        out_ref[...] = g_e * y

    @pl.when(jnp.logical_not(jnp.logical_and(e == 0, f == 0)))
    def _acc():
        out_ref[...] += g_e * y


def _moe_dense(xf, wg_pad, bg_pad, W1, W3, W2, *, block_t, n_f, interpret=False):
    n, h = xf.shape
    ff = W1.shape[-1]
    fb = ff // n_f
    n_t = n // block_t
    grid = (n_t, E, n_f)
    kernel = functools.partial(_moe_dense_kernel, n_e=E, n_f=n_f)
    return pl.pallas_call(
        kernel,
        grid=grid,
        in_specs=[
            pl.BlockSpec((block_t, h), lambda t, e, f: (t, 0)),
            pl.BlockSpec((h, LANES), lambda t, e, f: (0, 0)),
            pl.BlockSpec((1, LANES), lambda t, e, f: (0, 0)),
            pl.BlockSpec((1, h, fb), lambda t, e, f: (e, 0, f)),
            pl.BlockSpec((1, h, fb), lambda t, e, f: (e, 0, f)),
            pl.BlockSpec((1, fb, h), lambda t, e, f: (e, f, 0)),
        ],
        out_specs=pl.BlockSpec((block_t, h), lambda t, e, f: (t, 0)),
        out_shape=jax.ShapeDtypeStruct((n, h), jnp.float32),
        scratch_shapes=[pltpu.VMEM((block_t, LANES), jnp.float32)],
        compiler_params=pltpu.CompilerParams(
            dimension_semantics=("parallel", "arbitrary", "arbitrary"),
        ),
        interpret=interpret,
    )(xf, wg_pad, bg_pad, W1, W3, W2)


def kernel(x, Wg, bg, W1, W3, W2, interpret=False):
    B, S, H = x.shape
    n = B * S
    xf = x.reshape(n, H)
    wg_pad = jnp.zeros((H, LANES), jnp.float32).at[:, :E].set(Wg)
    bg_pad = jnp.zeros((1, LANES), jnp.float32).at[0, :E].set(bg)
    out = _moe_dense(xf, wg_pad, bg_pad, W1, W3, W2,
                     block_t=512, n_f=2, interpret=interpret)
    return out.reshape(B, S, H)
